# Initial kernel scaffold; baseline (speedup 1.0000x reference)
import jax
import jax.numpy as jnp
from jax.experimental import pallas as pl


def kernel(x, edge_index, W_conv, b_conv, W_lin, b_lin):
    raise NotImplementedError("write your pallas kernel here")


# speedup vs baseline: 39.5046x; 39.5046x over previous
import functools

import jax
import jax.numpy as jnp
from jax import lax
from jax.experimental import pallas as pl
from jax.experimental.pallas import tpu as pltpu
from jax.experimental.pallas import tpu_sc as plsc

NC = 2
NS = 16
NW = NC * NS
CHUNK = 128


def _sc_mesh():
    return plsc.VectorSubcoreMesh(core_axis_name="c", subcore_axis_name="s")


def _make_deg_kernel(chunks, npad, rows_per_tile):
    @functools.partial(
        pl.kernel,
        out_type=jax.ShapeDtypeStruct((NC, npad, 16), jnp.float32),
        mesh=_sc_mesh(),
        compiler_params=pltpu.CompilerParams(use_tc_tiling_on_sc=False),
        scratch_types=[
            pltpu.VMEM((chunks, CHUNK), jnp.int32),
            pltpu.VMEM((CHUNK, 16), jnp.float32),
            pltpu.VMEM_SHARED((npad, 16), jnp.float32),
        ],
    )
    def deg_kernel(dst_hbm, ones_hbm, zeros_hbm, out_hbm, dst_v, ones_v, deg_sh):
        c = lax.axis_index("c")
        s = lax.axis_index("s")
        base = s * rows_per_tile
        rows = pl.ds(base, rows_per_tile)
        pltpu.sync_copy(dst_hbm.at[c, s], dst_v)
        pltpu.sync_copy(ones_hbm, ones_v)
        pltpu.sync_copy(zeros_hbm.at[rows], deg_sh.at[rows])
        plsc.subcore_barrier()

        def body(j, carry):
            pltpu.sync_copy(ones_v, deg_sh.at[dst_v.at[j]], add=True)
            return carry

        lax.fori_loop(0, chunks, body, 0)
        plsc.subcore_barrier()
        pltpu.sync_copy(deg_sh.at[rows], out_hbm.at[c, rows])

    return deg_kernel


def _make_msg_kernel(chunks, npad, rows_per_tile, d_hid):
    @functools.partial(
        pl.kernel,
        out_type=jax.ShapeDtypeStruct((NC, npad, d_hid), jnp.float32),
        mesh=_sc_mesh(),
        compiler_params=pltpu.CompilerParams(use_tc_tiling_on_sc=False),
        scratch_types=[
            pltpu.VMEM((chunks, CHUNK), jnp.int32),
            pltpu.VMEM((chunks, CHUNK), jnp.int32),
            pltpu.VMEM((CHUNK, d_hid), jnp.float32),
            pltpu.VMEM_SHARED((npad, d_hid), jnp.float32),
        ],
    )
    def msg_kernel(src_hbm, dst_hbm, g_hbm, zeros_hbm, out_hbm,
                   src_v, dst_v, rows_v, acc_sh):
        c = lax.axis_index("c")
        s = lax.axis_index("s")
        base = s * rows_per_tile
        rows = pl.ds(base, rows_per_tile)
        pltpu.sync_copy(src_hbm.at[c, s], src_v)
        pltpu.sync_copy(dst_hbm.at[c, s], dst_v)
        pltpu.sync_copy(zeros_hbm.at[rows], acc_sh.at[rows])
        plsc.subcore_barrier()

        def body(j, carry):
            pltpu.sync_copy(g_hbm.at[src_v.at[j]], rows_v)
            pltpu.sync_copy(rows_v, acc_sh.at[dst_v.at[j]], add=True)
            return carry

        lax.fori_loop(0, chunks, body, 0)
        plsc.subcore_barrier()
        pltpu.sync_copy(acc_sh.at[rows], out_hbm.at[c, rows])

    return msg_kernel


def _transform_body(x_ref, w_ref, degp_ref, g_ref):
    h = jnp.dot(x_ref[...], w_ref[...], preferred_element_type=jnp.float32)
    deg = degp_ref[0] + degp_ref[1]
    dinv = lax.rsqrt(deg[:, 0:1] + 1.0)
    g_ref[...] = h * dinv


def _head_body(accp_ref, g_ref, degp_ref, bc_ref, wlt_ref, bl_ref, out_ref):
    deg = degp_ref[0] + degp_ref[1]
    dinv = lax.rsqrt(deg[:, 0:1] + 1.0)
    z = (accp_ref[0] + accp_ref[1] + g_ref[...]) * dinv + bc_ref[...]
    zr = jnp.maximum(z, 0.0)
    o = jnp.sum(zr * wlt_ref[...], axis=1, keepdims=True) + bl_ref[...]
    out_ref[...] = jax.nn.sigmoid(o)


def kernel(x, edge_index, W_conv, b_conv, W_lin, b_lin):
    n = x.shape[0]
    d_in = x.shape[1]
    d_hid = W_conv.shape[1]
    e = edge_index.shape[1]

    rows_per_tile = pl.cdiv(n, NS * 8) * 8
    npad = rows_per_tile * NS
    chunks = pl.cdiv(pl.cdiv(e, NW), CHUNK)
    epad = NW * chunks * CHUNK
    dummy = n

    src = edge_index[0].astype(jnp.int32)
    dst = edge_index[1].astype(jnp.int32)
    pad = jnp.full((epad - e,), dummy, jnp.int32)
    src4 = jnp.concatenate([src, pad]).reshape(NC, NS, chunks, CHUNK)
    dst4 = jnp.concatenate([dst, pad]).reshape(NC, NS, chunks, CHUNK)

    ones16 = jnp.ones((CHUNK, 16), jnp.float32)
    zeros16 = jnp.zeros((npad, 16), jnp.float32)
    zeros_hid = jnp.zeros((npad, d_hid), jnp.float32)
    x_pad = jnp.pad(x, ((0, npad - n), (0, 0)))

    degp = _make_deg_kernel(chunks, npad, rows_per_tile)(dst4, ones16, zeros16)

    nblk = NS
    g = pl.pallas_call(
        _transform_body,
        grid=(nblk,),
        in_specs=[
            pl.BlockSpec((rows_per_tile, d_in), lambda i: (i, 0)),
            pl.BlockSpec((d_in, d_hid), lambda i: (0, 0)),
            pl.BlockSpec((NC, rows_per_tile, 16), lambda i: (0, i, 0)),
        ],
        out_specs=pl.BlockSpec((rows_per_tile, d_hid), lambda i: (i, 0)),
        out_shape=jax.ShapeDtypeStruct((npad, d_hid), jnp.float32),
    )(x_pad, W_conv, degp)

    accp = _make_msg_kernel(chunks, npad, rows_per_tile, d_hid)(
        src4, dst4, g, zeros_hid)

    out_pad = pl.pallas_call(
        _head_body,
        grid=(nblk,),
        in_specs=[
            pl.BlockSpec((NC, rows_per_tile, d_hid), lambda i: (0, i, 0)),
            pl.BlockSpec((rows_per_tile, d_hid), lambda i: (i, 0)),
            pl.BlockSpec((NC, rows_per_tile, 16), lambda i: (0, i, 0)),
            pl.BlockSpec((1, d_hid), lambda i: (0, 0)),
            pl.BlockSpec((1, d_hid), lambda i: (0, 0)),
            pl.BlockSpec((1, 1), lambda i: (0, 0)),
        ],
        out_specs=pl.BlockSpec((rows_per_tile, 1), lambda i: (i, 0)),
        out_shape=jax.ShapeDtypeStruct((npad, 1), jnp.float32),
    )(accp, g, degp, b_conv.reshape(1, d_hid), W_lin.reshape(1, d_hid),
      b_lin.reshape(1, 1))

    return out_pad[:n]
